# baseline (device time: 52110 ns/iter reference)
import os

import jax
import jax.numpy as jnp
from jax import lax
from jax.experimental import pallas as pl
from jax.experimental.pallas import tpu as pltpu

N_DEV = 4
_ABLATE = os.environ.get("ABLATE", "")


def kernel(x, w_mat, scale_x, scale_w):
    m_total, k_shard = x.shape
    k_total, n = w_mat.shape
    m_per = m_total // N_DEV

    def body(x_ref, w_ref, sx_ref, sw_ref, out_ref,
             acc_ref, send_buf, recv_buf, w_vmem, send_sems, recv_sems,
             w_sems, out_sem):
        my = lax.axis_index("i")

        jseq = [my,
                lax.rem(my - 1 + N_DEV, N_DEV),
                lax.rem(my + 1, N_DEV),
                lax.rem(my + 2, N_DEV)]

        def w_copy(k):
            return pltpu.make_async_copy(
                w_ref.at[pl.ds(jseq[k] * m_per, m_per), :],
                w_vmem.at[k % 2],
                w_sems.at[k % 2],
            )

        w_copy(0).start()
        w_copy(1).start()

        with jax.named_scope("barrier"):
            barrier = pltpu.get_barrier_semaphore()
            for d in range(1, N_DEV):
                peer = lax.rem(my + d, N_DEV)
                pl.semaphore_signal(barrier, inc=1, device_id=(peer,),
                                    device_id_type=pl.DeviceIdType.MESH)
            pl.semaphore_wait(barrier, N_DEV - 1)

        rdmas = []
        with jax.named_scope("stage_send"):
            for k, dst in enumerate(
                    [lax.rem(my + 2, N_DEV),
                     lax.rem(my + 1, N_DEV),
                     lax.rem(my - 1 + N_DEV, N_DEV)]):
                send_buf[k] = x_ref[pl.ds(dst * m_per, m_per), :].astype(
                    jnp.float8_e4m3fn)
                rdma = pltpu.make_async_remote_copy(
                    src_ref=send_buf.at[k],
                    dst_ref=recv_buf.at[my],
                    send_sem=send_sems.at[k],
                    recv_sem=recv_sems.at[my],
                    device_id=(dst,),
                    device_id_type=pl.DeviceIdType.MESH,
                )
                if _ABLATE != "nocomm":
                    rdma.start()
                    rdmas.append(rdma)
            recv_buf[my] = x_ref[pl.ds(my * m_per, m_per), :].astype(
                jnp.float8_e4m3fn)

        dot = lambda a, b: lax.dot_general(
            a, b, (((1,), (0,)), ((), ())),
            preferred_element_type=jnp.float32)

        with jax.named_scope("local_dot"):
            w_copy(0).wait()
            acc_ref[...] = dot(recv_buf[my],
                               w_vmem[0].astype(jnp.float8_e5m2))
            w_copy(2).start()

        for d in range(1, N_DEV):
            src = jseq[d]
            recv = pltpu.make_async_remote_copy(
                src_ref=send_buf.at[0],
                dst_ref=recv_buf.at[src],
                send_sem=send_sems.at[0],
                recv_sem=recv_sems.at[src],
                device_id=(my,),
                device_id_type=pl.DeviceIdType.MESH,
            )
            with jax.named_scope(f"wait_recv#hop={d}"):
                if _ABLATE != "nocomm":
                    recv.wait_recv()
                w_copy(d).wait()
            with jax.named_scope(f"dot#hop={d}"):
                if _ABLATE != "nocompute":
                    acc_ref[...] += dot(recv_buf[src],
                                        w_vmem[d % 2].astype(jnp.float8_e5m2))
                if d + 2 < N_DEV:
                    w_copy(d + 2).start()

        with jax.named_scope("tail"):
            for rdma in rdmas:
                rdma.wait_send()

            acc_ref[...] = acc_ref[...] * (sx_ref[0] * sw_ref[0])
            out_cp = pltpu.make_async_copy(acc_ref, out_ref, out_sem)
            out_cp.start()
            out_cp.wait()

    return pl.pallas_call(
        body,
        out_shape=jax.ShapeDtypeStruct((m_per, n), jnp.float32),
        in_specs=[
            pl.BlockSpec(memory_space=pltpu.VMEM),
            pl.BlockSpec(memory_space=pl.ANY),
            pl.BlockSpec(memory_space=pltpu.SMEM),
            pl.BlockSpec(memory_space=pltpu.SMEM),
        ],
        out_specs=pl.BlockSpec(memory_space=pl.ANY),
        scratch_shapes=[
            pltpu.VMEM((m_per, n), jnp.float32),
            pltpu.VMEM((N_DEV - 1, m_per, k_shard), jnp.float8_e4m3fn),
            pltpu.VMEM((N_DEV, m_per, k_shard), jnp.float8_e4m3fn),
            pltpu.VMEM((2, m_per, n), jnp.float32),
            pltpu.SemaphoreType.DMA((N_DEV - 1,)),
            pltpu.SemaphoreType.DMA((N_DEV,)),
            pltpu.SemaphoreType.DMA((2,)),
            pltpu.SemaphoreType.DMA,
        ],
        compiler_params=pltpu.CompilerParams(
            collective_id=0, vmem_limit_bytes=100 * 1024 * 1024),
    )(x, w_mat, scale_x, scale_w)


# device time: 49948 ns/iter; 1.0433x vs baseline; 1.0433x over previous
import os

import jax
import jax.numpy as jnp
from jax import lax
from jax.experimental import pallas as pl
from jax.experimental.pallas import tpu as pltpu

N_DEV = 4
_ABLATE = os.environ.get("ABLATE", "")


def kernel(x, w_mat, scale_x, scale_w):
    m_total, k_shard = x.shape
    k_total, n = w_mat.shape
    m_per = m_total // N_DEV

    def body(x_ref, w_ref, sx_ref, sw_ref, out_ref,
             x_vmem, send_buf, recv_buf, w_vmem,
             x_sems, send_sems, recv_sems, w_sems):
        my = lax.axis_index("i")

        bseq = [lax.rem(my + 2, N_DEV),
                lax.rem(my + 1, N_DEV),
                lax.rem(my - 1 + N_DEV, N_DEV),
                my]

        def x_copy(k):
            return pltpu.make_async_copy(
                x_ref.at[pl.ds(bseq[k] * m_per, m_per), :],
                x_vmem.at[k % 2],
                x_sems.at[k % 2],
            )

        jseq = [my,
                lax.rem(my - 1 + N_DEV, N_DEV),
                lax.rem(my + 1, N_DEV),
                lax.rem(my + 2, N_DEV)]

        def w_copy(k):
            return pltpu.make_async_copy(
                w_ref.at[pl.ds(jseq[k] * m_per, m_per), :],
                w_vmem.at[k % 2],
                w_sems.at[k % 2],
            )

        x_copy(0).start()
        x_copy(1).start()
        w_copy(0).start()
        w_copy(1).start()

        with jax.named_scope("barrier"):
            barrier = pltpu.get_barrier_semaphore()
            for d in range(1, N_DEV):
                peer = lax.rem(my + d, N_DEV)
                pl.semaphore_signal(barrier, inc=1, device_id=(peer,),
                                    device_id_type=pl.DeviceIdType.MESH)
            pl.semaphore_wait(barrier, N_DEV - 1)

        rdmas = []
        with jax.named_scope("stage_send"):
            for k in range(3):
                x_copy(k).wait()
                send_buf[k] = x_vmem[k % 2].astype(jnp.float8_e4m3fn)
                if k + 2 < 4:
                    x_copy(k + 2).start()
                rdma = pltpu.make_async_remote_copy(
                    src_ref=send_buf.at[k],
                    dst_ref=recv_buf.at[my],
                    send_sem=send_sems.at[k],
                    recv_sem=recv_sems.at[my],
                    device_id=(bseq[k],),
                    device_id_type=pl.DeviceIdType.MESH,
                )
                if _ABLATE != "nocomm":
                    rdma.start()
                    rdmas.append(rdma)
            x_copy(3).wait()
            recv_buf[my] = x_vmem[1].astype(jnp.float8_e4m3fn)

        dot = lambda a, b: lax.dot_general(
            a, b, (((1,), (0,)), ((), ())),
            preferred_element_type=jnp.float32)

        with jax.named_scope("local_dot"):
            w_copy(0).wait()
            out_ref[...] = dot(recv_buf[my],
                               w_vmem[0].astype(jnp.float8_e5m2))
            w_copy(2).start()

        for d in range(1, N_DEV):
            src = jseq[d]
            recv = pltpu.make_async_remote_copy(
                src_ref=send_buf.at[0],
                dst_ref=recv_buf.at[src],
                send_sem=send_sems.at[0],
                recv_sem=recv_sems.at[src],
                device_id=(my,),
                device_id_type=pl.DeviceIdType.MESH,
            )
            with jax.named_scope(f"wait_recv#hop={d}"):
                if _ABLATE != "nocomm":
                    recv.wait_recv()
                w_copy(d).wait()
            with jax.named_scope(f"dot#hop={d}"):
                if _ABLATE != "nocompute":
                    out_ref[...] += dot(recv_buf[src],
                                        w_vmem[d % 2].astype(jnp.float8_e5m2))
                if d + 2 < N_DEV:
                    w_copy(d + 2).start()

        with jax.named_scope("tail"):
            for rdma in rdmas:
                rdma.wait_send()

            out_ref[...] = out_ref[...] * (sx_ref[0] * sw_ref[0])

    return pl.pallas_call(
        body,
        out_shape=jax.ShapeDtypeStruct((m_per, n), jnp.float32),
        in_specs=[
            pl.BlockSpec(memory_space=pl.ANY),
            pl.BlockSpec(memory_space=pl.ANY),
            pl.BlockSpec(memory_space=pltpu.SMEM),
            pl.BlockSpec(memory_space=pltpu.SMEM),
        ],
        out_specs=pl.BlockSpec(memory_space=pltpu.VMEM),
        scratch_shapes=[
            pltpu.VMEM((2, m_per, k_shard), jnp.float32),
            pltpu.VMEM((N_DEV - 1, m_per, k_shard), jnp.float8_e4m3fn),
            pltpu.VMEM((N_DEV, m_per, k_shard), jnp.float8_e4m3fn),
            pltpu.VMEM((2, m_per, n), jnp.float32),
            pltpu.SemaphoreType.DMA((2,)),
            pltpu.SemaphoreType.DMA((N_DEV - 1,)),
            pltpu.SemaphoreType.DMA((N_DEV,)),
            pltpu.SemaphoreType.DMA((2,)),
        ],
        compiler_params=pltpu.CompilerParams(
            collective_id=0, vmem_limit_bytes=100 * 1024 * 1024),
    )(x, w_mat, scale_x, scale_w)


# device time: 45336 ns/iter; 1.1494x vs baseline; 1.1017x over previous
import os

import jax
import jax.numpy as jnp
from jax import lax
from jax.experimental import pallas as pl
from jax.experimental.pallas import tpu as pltpu

N_DEV = 4
H = 2
_ABLATE = os.environ.get("ABLATE", "")


def kernel(x, w_mat, scale_x, scale_w):
    m_total, k_shard = x.shape
    k_total, n = w_mat.shape
    m_per = m_total // N_DEV
    m_h = m_per // H

    def body(x_ref, w_ref, sx_ref, sw_ref, out_ref,
             x_vmem, send_buf, recv_buf, w_vmem, w8_ref,
             x_sems, send_sems, recv_sems, w_sems):
        my = lax.axis_index("i")

        bseq = [lax.rem(my + 2, N_DEV),
                lax.rem(my + 1, N_DEV),
                lax.rem(my - 1 + N_DEV, N_DEV),
                my]
        jseq = [my,
                lax.rem(my - 1 + N_DEV, N_DEV),
                lax.rem(my + 1, N_DEV),
                lax.rem(my + 2, N_DEV)]

        def x_copy(s):
            k, h = divmod(s, H)
            return pltpu.make_async_copy(
                x_ref.at[pl.ds(bseq[k] * m_per + h * m_h, m_h), :],
                x_vmem.at[s % 2],
                x_sems.at[s % 2],
            )

        def w_copy(d):
            return pltpu.make_async_copy(
                w_ref.at[pl.ds(jseq[d] * m_per, m_per), :],
                w_vmem.at[d % 2],
                w_sems.at[d % 2],
            )

        x_copy(0).start()
        x_copy(1).start()
        w_copy(0).start()
        w_copy(1).start()

        with jax.named_scope("barrier"):
            barrier = pltpu.get_barrier_semaphore()
            for d in range(1, N_DEV):
                peer = lax.rem(my + d, N_DEV)
                pl.semaphore_signal(barrier, inc=1, device_id=(peer,),
                                    device_id_type=pl.DeviceIdType.MESH)
            pl.semaphore_wait(barrier, N_DEV - 1)

        rdmas = []
        with jax.named_scope("stage_send"):
            for s in range(3 * H):
                k, h = divmod(s, H)
                x_copy(s).wait()
                send_buf[k, pl.ds(h * m_h, m_h), :] = x_vmem[s % 2].astype(
                    jnp.float8_e4m3fn)
                x_copy(s + 2).start()
                rdma = pltpu.make_async_remote_copy(
                    src_ref=send_buf.at[k, pl.ds(h * m_h, m_h), :],
                    dst_ref=recv_buf.at[my, pl.ds(h * m_h, m_h), :],
                    send_sem=send_sems.at[k, h],
                    recv_sem=recv_sems.at[my, h],
                    device_id=(bseq[k],),
                    device_id_type=pl.DeviceIdType.MESH,
                )
                if _ABLATE != "nocomm":
                    rdma.start()
                    rdmas.append(rdma)
            for s in (3 * H, 3 * H + 1):
                h = s % H
                x_copy(s).wait()
                recv_buf[my, pl.ds(h * m_h, m_h), :] = x_vmem[s % 2].astype(
                    jnp.float8_e4m3fn)

        dot = lambda a, b: lax.dot_general(
            a, b, (((1,), (0,)), ((), ())),
            preferred_element_type=jnp.float32)

        def w8(j):
            return w8_ref[pl.ds(j * m_per, m_per), :]

        for d in range(N_DEV):
            with jax.named_scope(f"wconv#blk={d}"):
                w_copy(d).wait()
                w8_ref[pl.ds(jseq[d] * m_per, m_per), :] = w_vmem[
                    d % 2].astype(jnp.float8_e5m2)
                if d + 2 < N_DEV:
                    w_copy(d + 2).start()
            if d == 0:
                with jax.named_scope("local_dot"):
                    out_ref[...] = dot(recv_buf[my], w8(my))

        for d in range(1, N_DEV):
            src = jseq[d]
            for h in range(H):
                recv = pltpu.make_async_remote_copy(
                    src_ref=send_buf.at[0, pl.ds(0, m_h), :],
                    dst_ref=recv_buf.at[src, pl.ds(h * m_h, m_h), :],
                    send_sem=send_sems.at[0, 0],
                    recv_sem=recv_sems.at[src, h],
                    device_id=(my,),
                    device_id_type=pl.DeviceIdType.MESH,
                )
                with jax.named_scope(f"wait_recv#hop={d}_{h}"):
                    if _ABLATE != "nocomm":
                        recv.wait_recv()
                with jax.named_scope(f"dot#hop={d}_{h}"):
                    if _ABLATE != "nocompute":
                        out_ref[pl.ds(h * m_h, m_h), :] += dot(
                            recv_buf[src, pl.ds(h * m_h, m_h), :], w8(src))

        with jax.named_scope("tail"):
            for rdma in rdmas:
                rdma.wait_send()

            out_ref[...] = out_ref[...] * (sx_ref[0] * sw_ref[0])

    return pl.pallas_call(
        body,
        out_shape=jax.ShapeDtypeStruct((m_per, n), jnp.float32),
        in_specs=[
            pl.BlockSpec(memory_space=pl.ANY),
            pl.BlockSpec(memory_space=pl.ANY),
            pl.BlockSpec(memory_space=pltpu.SMEM),
            pl.BlockSpec(memory_space=pltpu.SMEM),
        ],
        out_specs=pl.BlockSpec(memory_space=pltpu.VMEM),
        scratch_shapes=[
            pltpu.VMEM((2, m_h, k_shard), jnp.float32),
            pltpu.VMEM((N_DEV - 1, m_per, k_shard), jnp.float8_e4m3fn),
            pltpu.VMEM((N_DEV, m_per, k_shard), jnp.float8_e4m3fn),
            pltpu.VMEM((2, m_per, n), jnp.float32),
            pltpu.VMEM((k_total, n), jnp.float8_e5m2),
            pltpu.SemaphoreType.DMA((2,)),
            pltpu.SemaphoreType.DMA((N_DEV - 1, H)),
            pltpu.SemaphoreType.DMA((N_DEV, H)),
            pltpu.SemaphoreType.DMA((2,)),
        ],
        compiler_params=pltpu.CompilerParams(
            collective_id=0, vmem_limit_bytes=100 * 1024 * 1024),
    )(x, w_mat, scale_x, scale_w)
